# baseline (speedup 1.0000x reference)
import jax
import jax.numpy as jnp
from jax.experimental import pallas as pl
from jax.experimental.pallas import tpu as pltpu

_BETA = 0.25
_N_TOK = 2048
_CODE_DIM = 256
_K = 1024
_BLK = 256


def _vq_block(z_ref, c_hbm, zq_ref, part_ref, c_s, c2_s, cbf_s, sem):
    @pl.when(pl.program_id(0) == 0)
    def _init():
        copy = pltpu.make_async_copy(c_hbm, c_s, sem)
        copy.start()
        copy.wait()
        c0 = c_s[...]
        c2_s[...] = jnp.sum(c0 * c0, axis=1)[None, :]
        cbf_s[...] = c0.astype(jnp.bfloat16)

    c = c_s[...]
    c2 = c2_s[...]
    cbf = cbf_s[...]
    z = z_ref[...]
    m = jnp.dot(z, c.T, preferred_element_type=jnp.float32)
    z2 = jnp.sum(z * z, axis=1, keepdims=True)
    dist = z2 - 2.0 * m + c2
    rowmin = jnp.min(dist, axis=1, keepdims=True)
    iota = jax.lax.broadcasted_iota(jnp.int32, dist.shape, 1)
    idx = jnp.min(jnp.where(dist == rowmin, iota, _K), axis=1,
                  keepdims=True)
    onehot = (iota == idx).astype(jnp.bfloat16)
    zq = jnp.dot(onehot, cbf, preferred_element_type=jnp.float32)
    zq_ref[...] = zq
    part_ref[...] = jnp.full((1, 1, 128), jnp.sum(rowmin), jnp.float32)


def kernel(z, codebook):
    z = z.reshape(z.shape[0], -1)
    zq, parts = pl.pallas_call(
        _vq_block,
        grid=(_N_TOK // _BLK,),
        in_specs=[
            pl.BlockSpec((_BLK, _CODE_DIM), lambda i: (i, 0)),
            pl.BlockSpec(memory_space=pltpu.MemorySpace.HBM),
        ],
        out_specs=[
            pl.BlockSpec((_BLK, _CODE_DIM), lambda i: (i, 0)),
            pl.BlockSpec((1, 1, 128), lambda i: (i, 0, 0)),
        ],
        out_shape=[
            jax.ShapeDtypeStruct((_N_TOK, _CODE_DIM), jnp.float32),
            jax.ShapeDtypeStruct((_N_TOK // _BLK, 1, 128), jnp.float32),
        ],
        scratch_shapes=[
            pltpu.VMEM((_K, _CODE_DIM), jnp.float32),
            pltpu.VMEM((1, _K), jnp.float32),
            pltpu.VMEM((_K, _CODE_DIM), jnp.bfloat16),
            pltpu.SemaphoreType.DMA,
        ],
    )(z, codebook)
    mean_sq = jnp.sum(parts[:, 0, 0]) / (_N_TOK * _CODE_DIM)
    loss = _BETA * mean_sq + mean_sq
    return (zq, loss)


# speedup vs baseline: 1.8334x; 1.8334x over previous
import jax
import jax.numpy as jnp
from jax.experimental import pallas as pl

_BETA = 0.25
_N_TOK = 2048
_CODE_DIM = 256
_K = 1024
_BLK = 256


def _vq_kernel(z_ref, c_ref, zq_ref, part_ref):
    c = c_ref[...]
    c2 = jnp.sum(c * c, axis=1)[None, :]
    cbf = c.astype(jnp.bfloat16)
    total = jnp.zeros((), jnp.float32)
    for h in range(_N_TOK // _BLK):
        z = z_ref[pl.ds(h * _BLK, _BLK), :]
        m = jnp.dot(z, c.T, preferred_element_type=jnp.float32)
        z2 = jnp.sum(z * z, axis=1, keepdims=True)
        dist = z2 - 2.0 * m + c2
        rowmin = jnp.min(dist, axis=1, keepdims=True)
        iota = jax.lax.broadcasted_iota(jnp.int32, dist.shape, 1).astype(jnp.float32)
        idx = jnp.min(jnp.where(dist == rowmin, iota, float(_K)), axis=1,
                      keepdims=True)
        onehot = (iota == idx).astype(jnp.bfloat16)
        zq = jnp.dot(onehot, cbf, preferred_element_type=jnp.float32)
        zq_ref[pl.ds(h * _BLK, _BLK), :] = zq
        total = total + jnp.sum(rowmin)
    part_ref[...] = jnp.full((1, 128), total, jnp.float32)


def kernel(z, codebook):
    z = z.reshape(z.shape[0], -1)
    zq, parts = pl.pallas_call(
        _vq_kernel,
        out_shape=[
            jax.ShapeDtypeStruct((_N_TOK, _CODE_DIM), jnp.float32),
            jax.ShapeDtypeStruct((1, 128), jnp.float32),
        ],
    )(z, codebook)
    mean_sq = parts[0, 0] / (_N_TOK * _CODE_DIM)
    loss = _BETA * mean_sq + mean_sq
    return (zq, loss)
